# baseline (speedup 1.0000x reference)
import functools

import jax
import jax.numpy as jnp
from jax import lax
from jax.experimental import pallas as pl
from jax.experimental.pallas import tpu as pltpu
from jax.experimental.pallas import tpu_sc as plsc

NS = 16
NL = 16
K = 128
BM = 1000
NH = 3456
NP = 3


def _mmt(a, w):
    return lax.dot_general(a, w, (((1,), (1,)), ((), ())),
                           preferred_element_type=jnp.float32)


def _cnt_pad(n_nodes):
    gran = NS * NL
    return -(-(n_nodes + 1) // gran) * gran


def _make_segsum(n_nodes, d, nch, with_counts):
    stripe = NH // NS
    npad = _cnt_pad(n_nodes)
    cpt = npad // NS

    if with_counts:
        outs = [jax.ShapeDtypeStruct((NP * NH, d), jnp.float32),
                jax.ShapeDtypeStruct((NP * NH, d), jnp.float32),
                jax.ShapeDtypeStruct((2 * NS * npad,), jnp.float32)]
    else:
        outs = [jax.ShapeDtypeStruct((NP * NH, d), jnp.float32)]
    scratch = [
        pltpu.VMEM((nch, K), jnp.int32),
        pltpu.VMEM((nch, K), jnp.int32),
        pltpu.VMEM((K, d), jnp.float32),
        pltpu.VMEM((K, d), jnp.float32),
        pltpu.VMEM((K, d), jnp.float32),
        pltpu.VMEM_SHARED((NH + 8, d), jnp.float32),
    ]
    if with_counts:
        scratch += [
            pltpu.VMEM((npad,), jnp.float32),
        ]
    scratch += [pltpu.SemaphoreType.DMA] * 6

    mesh = plsc.VectorSubcoreMesh(core_axis_name="c", subcore_axis_name="s",
                                  num_cores=1, num_subcores=NS)

    @functools.partial(
        pl.kernel, out_type=tuple(outs), mesh=mesh, scratch_types=scratch,
        compiler_params=pltpu.CompilerParams(needs_layout_passes=False))
    def seg(*refs):
        if with_counts:
            (x0, src0, dstp0, dstg0, x1, src1, dstp1, dstg1, zf, zc,
             pout0, pout1, craw, srcv, dstv, r0, r1, r2, accum,
             cntv, *sems) = refs
            types = ((x0, src0, dstp0, dstg0, pout0),
                     (x1, src1, dstp1, dstg1, pout1))
        else:
            (x0, src0, dstp0, zf, pout0,
             srcv, dstv, r0, r1, r2, accum, *sems) = refs
            types = ((x0, src0, dstp0, None, pout0),)
        s = lax.axis_index("s")
        rows = (r0, r1, r2)
        gsem = sems[0:3]
        ssem = sems[3:6]

        def wait_g(sl):
            pltpu.make_async_copy(x0.at[srcv.at[0]], rows[sl],
                                  gsem[sl]).wait()

        def wait_s(sl):
            pltpu.make_async_copy(rows[sl], accum.at[dstv.at[0]],
                                  ssem[sl]).wait()

        for t, (x, src, dstp, dstg, pout) in enumerate(types):
            pltpu.sync_copy(src.at[s], srcv)
            for p in range(NP):
                pltpu.sync_copy(dstp.at[p, s], dstv)
                pltpu.sync_copy(zf, accum.at[pl.ds(s * stripe, stripe)])

                @pl.when(s == 0)
                def _():
                    pltpu.sync_copy(zf.at[pl.ds(0, 8)],
                                    accum.at[pl.ds(NH, 8)])

                plsc.subcore_barrier()

                pltpu.async_copy(x.at[srcv.at[0]], rows[0], gsem[0])
                pltpu.async_copy(x.at[srcv.at[1]], rows[1], gsem[1])
                wait_g(0)
                pltpu.async_copy(rows[0], accum.at[dstv.at[0]], ssem[0],
                                 add=True)
                pltpu.async_copy(x.at[srcv.at[2]], rows[2], gsem[2])

                @pl.loop(1, nch - 2, step=3)
                def _(base):
                    for j in range(3):
                        c = base + j
                        sl = (1 + j) % 3
                        wait_s(j)
                        pltpu.async_copy(x.at[srcv.at[c + 2]], rows[j],
                                         gsem[j])
                        wait_g(sl)
                        pltpu.async_copy(rows[sl], accum.at[dstv.at[c]],
                                         ssem[sl], add=True)

                wait_s((nch - 3) % 3)
                wait_g((nch - 2) % 3)
                pltpu.async_copy(rows[(nch - 2) % 3],
                                 accum.at[dstv.at[nch - 2]],
                                 ssem[(nch - 2) % 3], add=True)
                wait_g((nch - 1) % 3)
                pltpu.async_copy(rows[(nch - 1) % 3],
                                 accum.at[dstv.at[nch - 1]],
                                 ssem[(nch - 1) % 3], add=True)
                wait_s((nch - 2) % 3)
                wait_s((nch - 1) % 3)
                plsc.subcore_barrier()
                pltpu.sync_copy(accum.at[pl.ds(s * stripe, stripe)],
                                pout.at[pl.ds(p * NH + s * stripe, stripe)])

            if with_counts:
                pltpu.sync_copy(zc, cntv)
                pltpu.sync_copy(dstg.at[s], dstv)
                ones = jnp.full((NL,), 1.0, jnp.float32)

                @pl.loop(0, nch)
                def _(ci):
                    for i in range(K // NL):
                        idx16 = dstv[ci, pl.ds(i * NL, NL)]
                        plsc.addupdate_scatter(cntv, [idx16], ones)

                pltpu.sync_copy(
                    cntv, craw.at[pl.ds((t * NS + s) * npad, npad)])

    return seg


def _make_cnt_reduce(n_nodes):
    npad = _cnt_pad(n_nodes)
    cpt = npad // NS

    mesh = plsc.VectorSubcoreMesh(core_axis_name="c", subcore_axis_name="s",
                                  num_cores=1, num_subcores=NS)

    @functools.partial(
        pl.kernel,
        out_type=jax.ShapeDtypeStruct((2 * npad,), jnp.float32),
        mesh=mesh,
        scratch_types=[
            pltpu.VMEM((NS, cpt), jnp.float32),
            pltpu.VMEM((cpt,), jnp.float32),
            pltpu.SemaphoreType.DMA,
        ],
        compiler_params=pltpu.CompilerParams(needs_layout_passes=False))
    def red(craw, cout, cstripe, cred, sem):
        s = lax.axis_index("s")
        for t in range(2):
            descs = [
                pltpu.async_copy(
                    craw.at[pl.ds((t * NS + r) * npad + s * cpt, cpt)],
                    cstripe.at[r], sem)
                for r in range(NS)
            ]
            for de in descs:
                de.wait()
            for j in range(cpt // NL):
                acc = cstripe[0, pl.ds(j * NL, NL)]
                for r in range(1, NS):
                    acc = acc + cstripe[r, pl.ds(j * NL, NL)]
                cred[pl.ds(j * NL, NL)] = acc
            pltpu.sync_copy(cred, cout.at[pl.ds(t * npad + s * cpt, cpt)])

    return red


def _prep_edges(src, dst, nch, n_nodes):
    tot = NS * nch * K
    padn = tot - src.shape[0]
    if padn:
        src = jnp.concatenate([src, jnp.zeros((padn,), jnp.int32)])
        dst = jnp.concatenate([dst, jnp.full((padn,), n_nodes, jnp.int32)])
    shp = (NS, nch, K)
    dstp = jnp.stack([
        jnp.where((dst >= p * NH) & (dst < min((p + 1) * NH, n_nodes)),
                  dst - p * NH, NH).reshape(shp)
        for p in range(NP)
    ])
    return src.reshape(shp), dstp, dst.reshape(shp)


def kernel(x_students, x_assignments, ei_sa, ei_as, Wp_s, bp_s, Wp_a, bp_a,
           Wl_sa_0, bl_sa_0, Wr_sa_0, Wl_as_0, bl_as_0, Wr_as_0,
           Wl_sa_1, bl_sa_1, Wr_sa_1, Wl_as_1, bl_as_1, Wr_as_1,
           Wo, bo):
    n, d = x_students.shape
    e = ei_sa.shape[1]
    nb = n // BM
    npad = _cnt_pad(n)

    def proj_body(xs_ref, xa_ref, ws_ref, wa_ref, bs_ref, ba_ref,
                  os_ref, oa_ref):
        os_ref[...] = jnp.maximum(_mmt(xs_ref[...], ws_ref[...]) + bs_ref[...],
                                  0.0)
        oa_ref[...] = jnp.maximum(_mmt(xa_ref[...], wa_ref[...]) + ba_ref[...],
                                  0.0)

    wspec = pl.BlockSpec((d, d), lambda i: (0, 0))
    bspec = pl.BlockSpec((1, d), lambda i: (0, 0))
    rowspec = pl.BlockSpec((BM, d), lambda i: (i, 0))
    fspec = jax.ShapeDtypeStruct((n, d), jnp.float32)
    xs, xa = pl.pallas_call(
        proj_body,
        grid=(nb,),
        in_specs=[rowspec, rowspec, wspec, wspec, bspec, bspec],
        out_specs=[rowspec, rowspec],
        out_shape=[fspec, fspec],
    )(x_students, x_assignments, Wp_s, Wp_a,
      bp_s.reshape(1, d), bp_a.reshape(1, d))

    zf = jnp.zeros((NH // NS, d), jnp.float32)
    zc = jnp.zeros((npad,), jnp.float32)

    nch = -(-e // (NS * K))
    nch += (-nch) % 3
    src_sa, dstp_sa, dstg_sa = _prep_edges(ei_sa[0], ei_sa[1], nch, n)
    src_as, dstp_as, dstg_as = _prep_edges(ei_as[0], ei_as[1], nch, n)
    seg_c = _make_segsum(n, d, nch, with_counts=True)
    p_sa, p_as, craw = seg_c(
        xs, src_sa, dstp_sa, dstg_sa, xa, src_as, dstp_as, dstg_as, zf, zc)
    c_all = _make_cnt_reduce(n)(craw)
    cnt_sa = jnp.broadcast_to(c_all[:n, None], (n, d))
    cnt_as = jnp.broadcast_to(c_all[npad:npad + n, None], (n, d))

    def l0_body(psa_ref, csa_ref, pas_ref, cas_ref, xs_ref, xa_ref,
                wl_sa, wr_sa, wl_as, wr_as, bsa_ref, bas_ref,
                xa1_ref, xs1_ref):
        mean_sa = psa_ref[...] / jnp.maximum(csa_ref[...], 1.0)
        xa1_ref[...] = jnp.maximum(
            _mmt(mean_sa, wl_sa[...]) + bsa_ref[...]
            + _mmt(xa_ref[...], wr_sa[...]), 0.0)
        mean_as = pas_ref[...] / jnp.maximum(cas_ref[...], 1.0)
        xs1_ref[...] = jnp.maximum(
            _mmt(mean_as, wl_as[...]) + bas_ref[...]
            + _mmt(xs_ref[...], wr_as[...]), 0.0)

    xa1, xs1 = pl.pallas_call(
        l0_body,
        grid=(nb,),
        in_specs=[rowspec, rowspec, rowspec, rowspec, rowspec, rowspec,
                  wspec, wspec, wspec, wspec, bspec, bspec],
        out_specs=[rowspec, rowspec],
        out_shape=[fspec, fspec],
    )(p_sa, cnt_sa, p_as, cnt_as, xs, xa, Wl_sa_0, Wr_sa_0, Wl_as_0, Wr_as_0,
      bl_sa_0.reshape(1, d), bl_as_0.reshape(1, d))

    seg_n = _make_segsum(n, d, nch, with_counts=False)
    p1 = seg_n(xs1, src_sa, dstp_sa, zf)
    if isinstance(p1, (tuple, list)):
        p1 = p1[0]

    def l1_body(p_ref, c_ref, xa1_ref, wl, wr, bl_ref, wo_ref, bo_ref, o_ref):
        mean = p_ref[...] / jnp.maximum(c_ref[...], 1.0)
        h = jnp.maximum(
            _mmt(mean, wl[...]) + bl_ref[...] + _mmt(xa1_ref[...], wr[...]),
            0.0)
        o_ref[...] = (jnp.sum(h * wo_ref[...], axis=1, keepdims=True)
                      + bo_ref[0, 0])

    pred = pl.pallas_call(
        l1_body,
        grid=(nb,),
        in_specs=[
            rowspec, rowspec, rowspec, wspec, wspec, bspec,
            pl.BlockSpec((1, d), lambda i: (0, 0)),
            pl.BlockSpec((1, 1), lambda i: (0, 0)),
        ],
        out_specs=pl.BlockSpec((BM, 1), lambda i: (i, 0)),
        out_shape=jax.ShapeDtypeStruct((n, 1), jnp.float32),
    )(p1, cnt_sa, xa1, Wl_sa_1, Wr_sa_1, bl_sa_1.reshape(1, d),
      Wo, bo.reshape(1, 1))

    return pred[:, 0]


# speedup vs baseline: 1.2064x; 1.2064x over previous
import functools

import jax
import jax.numpy as jnp
from jax import lax
from jax.experimental import pallas as pl
from jax.experimental.pallas import tpu as pltpu
from jax.experimental.pallas import tpu_sc as plsc

NS = 16
NL = 16
K = 128
BM = 1000
NH = 3456
NP = 3


def _mmt(a, w):
    return lax.dot_general(a, w, (((1,), (1,)), ((), ())),
                           preferred_element_type=jnp.float32)


def _cnt_pad(n_nodes):
    gran = NS * NL
    return -(-(n_nodes + 1) // gran) * gran


def _make_segsum(n_nodes, d, nch, with_counts):
    stripe = NH // NS
    npad = _cnt_pad(n_nodes)
    cpt = npad // NS

    if with_counts:
        outs = [jax.ShapeDtypeStruct((NP * NH, d), jnp.float32),
                jax.ShapeDtypeStruct((NP * NH, d), jnp.float32),
                jax.ShapeDtypeStruct((2 * NS * npad,), jnp.float32)]
    else:
        outs = [jax.ShapeDtypeStruct((NP * NH, d), jnp.float32)]
    scratch = [
        pltpu.VMEM((nch, K), jnp.int32),
        pltpu.VMEM((nch, K), jnp.int32),
        pltpu.VMEM((K, d), jnp.float32),
        pltpu.VMEM((K, d), jnp.float32),
        pltpu.VMEM((K, d), jnp.float32),
        pltpu.VMEM_SHARED((NH + 8, d), jnp.float32),
    ]
    if with_counts:
        scratch += [
            pltpu.VMEM((npad,), jnp.float32),
        ]
    scratch += [pltpu.SemaphoreType.DMA] * 6

    mesh = plsc.VectorSubcoreMesh(core_axis_name="c", subcore_axis_name="s",
                                  num_cores=1, num_subcores=NS)

    @functools.partial(
        pl.kernel, out_type=tuple(outs), mesh=mesh, scratch_types=scratch,
        compiler_params=pltpu.CompilerParams(needs_layout_passes=False))
    def seg(*refs):
        if with_counts:
            (x0, src0, dstp0, dstg0, x1, src1, dstp1, dstg1, zf, zc,
             pout0, pout1, craw, srcv, dstv, r0, r1, r2, accum,
             cntv, *sems) = refs
            types = ((x0, src0, dstp0, dstg0, pout0),
                     (x1, src1, dstp1, dstg1, pout1))
        else:
            (x0, src0, dstp0, zf, pout0,
             srcv, dstv, r0, r1, r2, accum, *sems) = refs
            types = ((x0, src0, dstp0, None, pout0),)
        s = lax.axis_index("s")
        rows = (r0, r1, r2)
        gsem = sems[0:3]
        ssem = sems[3:6]

        def wait_g(sl):
            pltpu.make_async_copy(x0.at[srcv.at[0]], rows[sl],
                                  gsem[sl]).wait()

        def wait_s(sl):
            pltpu.make_async_copy(rows[sl], accum.at[dstv.at[0]],
                                  ssem[sl]).wait()

        for t, (x, src, dstp, dstg, pout) in enumerate(types):
            pltpu.sync_copy(src.at[s], srcv)
            for p in range(NP):
                pltpu.sync_copy(dstp.at[p, s], dstv)
                pltpu.sync_copy(zf, accum.at[pl.ds(s * stripe, stripe)])

                @pl.when(s == 0)
                def _():
                    pltpu.sync_copy(zf.at[pl.ds(0, 8)],
                                    accum.at[pl.ds(NH, 8)])

                plsc.subcore_barrier()

                pltpu.async_copy(x.at[srcv.at[0]], rows[0], gsem[0])
                pltpu.async_copy(x.at[srcv.at[1]], rows[1], gsem[1])
                wait_g(0)
                pltpu.async_copy(rows[0], accum.at[dstv.at[0]], ssem[0],
                                 add=True)
                pltpu.async_copy(x.at[srcv.at[2]], rows[2], gsem[2])

                @pl.loop(1, nch - 2, step=3)
                def _(base):
                    for j in range(3):
                        c = base + j
                        sl = (1 + j) % 3
                        wait_s(j)
                        pltpu.async_copy(x.at[srcv.at[c + 2]], rows[j],
                                         gsem[j])
                        wait_g(sl)
                        pltpu.async_copy(rows[sl], accum.at[dstv.at[c]],
                                         ssem[sl], add=True)

                wait_s((nch - 3) % 3)
                wait_g((nch - 2) % 3)
                pltpu.async_copy(rows[(nch - 2) % 3],
                                 accum.at[dstv.at[nch - 2]],
                                 ssem[(nch - 2) % 3], add=True)
                wait_g((nch - 1) % 3)
                pltpu.async_copy(rows[(nch - 1) % 3],
                                 accum.at[dstv.at[nch - 1]],
                                 ssem[(nch - 1) % 3], add=True)
                wait_s((nch - 2) % 3)
                wait_s((nch - 1) % 3)
                plsc.subcore_barrier()
                pltpu.sync_copy(accum.at[pl.ds(s * stripe, stripe)],
                                pout.at[pl.ds(p * NH + s * stripe, stripe)])

            if with_counts:
                pltpu.sync_copy(zc, cntv)
                pltpu.sync_copy(dstg.at[s], dstv)
                ones = jnp.full((NL,), 1.0, jnp.float32)

                @pl.loop(0, nch)
                def _(ci):
                    for i in range(K // NL):
                        idx16 = dstv[ci, pl.ds(i * NL, NL)]
                        plsc.addupdate_scatter(cntv, [idx16], ones)

                pltpu.sync_copy(
                    cntv, craw.at[pl.ds((t * NS + s) * npad, npad)])

    return seg


def _make_cnt_reduce(n_nodes):
    npad = _cnt_pad(n_nodes)
    cpt = npad // NS

    mesh = plsc.VectorSubcoreMesh(core_axis_name="c", subcore_axis_name="s",
                                  num_cores=1, num_subcores=NS)

    @functools.partial(
        pl.kernel,
        out_type=jax.ShapeDtypeStruct((2 * npad,), jnp.float32),
        mesh=mesh,
        scratch_types=[
            pltpu.VMEM((NS, cpt), jnp.float32),
            pltpu.VMEM((cpt,), jnp.float32),
            pltpu.SemaphoreType.DMA,
        ],
        compiler_params=pltpu.CompilerParams(needs_layout_passes=False))
    def red(craw, cout, cstripe, cred, sem):
        s = lax.axis_index("s")
        for t in range(2):
            descs = [
                pltpu.async_copy(
                    craw.at[pl.ds((t * NS + r) * npad + s * cpt, cpt)],
                    cstripe.at[r], sem)
                for r in range(NS)
            ]
            for de in descs:
                de.wait()
            for j in range(cpt // NL):
                acc = cstripe[0, pl.ds(j * NL, NL)]
                for r in range(1, NS):
                    acc = acc + cstripe[r, pl.ds(j * NL, NL)]
                cred[pl.ds(j * NL, NL)] = acc
            pltpu.sync_copy(cred, cout.at[pl.ds(t * npad + s * cpt, cpt)])

    return red


def _prep_edges(src, dst, nch, n_nodes):
    tot = NS * nch * K
    padn = tot - src.shape[0]
    if padn:
        src = jnp.concatenate([src, jnp.zeros((padn,), jnp.int32)])
        dst = jnp.concatenate([dst, jnp.full((padn,), n_nodes, jnp.int32)])
    shp = (NS, nch, K)
    dummy = NH + (jnp.arange(tot, dtype=jnp.int32) & 7)
    dstp = jnp.stack([
        jnp.where((dst >= p * NH) & (dst < min((p + 1) * NH, n_nodes)),
                  dst - p * NH, dummy).reshape(shp)
        for p in range(NP)
    ])
    return src.reshape(shp), dstp, dst.reshape(shp)


def kernel(x_students, x_assignments, ei_sa, ei_as, Wp_s, bp_s, Wp_a, bp_a,
           Wl_sa_0, bl_sa_0, Wr_sa_0, Wl_as_0, bl_as_0, Wr_as_0,
           Wl_sa_1, bl_sa_1, Wr_sa_1, Wl_as_1, bl_as_1, Wr_as_1,
           Wo, bo):
    n, d = x_students.shape
    e = ei_sa.shape[1]
    nb = n // BM
    npad = _cnt_pad(n)

    def proj_body(xs_ref, xa_ref, ws_ref, wa_ref, bs_ref, ba_ref,
                  os_ref, oa_ref):
        os_ref[...] = jnp.maximum(_mmt(xs_ref[...], ws_ref[...]) + bs_ref[...],
                                  0.0)
        oa_ref[...] = jnp.maximum(_mmt(xa_ref[...], wa_ref[...]) + ba_ref[...],
                                  0.0)

    wspec = pl.BlockSpec((d, d), lambda i: (0, 0))
    bspec = pl.BlockSpec((1, d), lambda i: (0, 0))
    rowspec = pl.BlockSpec((BM, d), lambda i: (i, 0))
    fspec = jax.ShapeDtypeStruct((n, d), jnp.float32)
    xs, xa = pl.pallas_call(
        proj_body,
        grid=(nb,),
        in_specs=[rowspec, rowspec, wspec, wspec, bspec, bspec],
        out_specs=[rowspec, rowspec],
        out_shape=[fspec, fspec],
    )(x_students, x_assignments, Wp_s, Wp_a,
      bp_s.reshape(1, d), bp_a.reshape(1, d))

    zf = jnp.zeros((NH // NS, d), jnp.float32)
    zc = jnp.zeros((npad,), jnp.float32)

    nch = -(-e // (NS * K))
    nch += (-nch) % 3
    src_sa, dstp_sa, dstg_sa = _prep_edges(ei_sa[0], ei_sa[1], nch, n)
    src_as, dstp_as, dstg_as = _prep_edges(ei_as[0], ei_as[1], nch, n)
    seg_c = _make_segsum(n, d, nch, with_counts=True)
    p_sa, p_as, craw = seg_c(
        xs, src_sa, dstp_sa, dstg_sa, xa, src_as, dstp_as, dstg_as, zf, zc)
    c_all = _make_cnt_reduce(n)(craw)
    cnt_sa = jnp.broadcast_to(c_all[:n, None], (n, d))
    cnt_as = jnp.broadcast_to(c_all[npad:npad + n, None], (n, d))

    def l0_body(psa_ref, csa_ref, pas_ref, cas_ref, xs_ref, xa_ref,
                wl_sa, wr_sa, wl_as, wr_as, bsa_ref, bas_ref,
                xa1_ref, xs1_ref):
        mean_sa = psa_ref[...] / jnp.maximum(csa_ref[...], 1.0)
        xa1_ref[...] = jnp.maximum(
            _mmt(mean_sa, wl_sa[...]) + bsa_ref[...]
            + _mmt(xa_ref[...], wr_sa[...]), 0.0)
        mean_as = pas_ref[...] / jnp.maximum(cas_ref[...], 1.0)
        xs1_ref[...] = jnp.maximum(
            _mmt(mean_as, wl_as[...]) + bas_ref[...]
            + _mmt(xs_ref[...], wr_as[...]), 0.0)

    xa1, xs1 = pl.pallas_call(
        l0_body,
        grid=(nb,),
        in_specs=[rowspec, rowspec, rowspec, rowspec, rowspec, rowspec,
                  wspec, wspec, wspec, wspec, bspec, bspec],
        out_specs=[rowspec, rowspec],
        out_shape=[fspec, fspec],
    )(p_sa, cnt_sa, p_as, cnt_as, xs, xa, Wl_sa_0, Wr_sa_0, Wl_as_0, Wr_as_0,
      bl_sa_0.reshape(1, d), bl_as_0.reshape(1, d))

    seg_n = _make_segsum(n, d, nch, with_counts=False)
    p1 = seg_n(xs1, src_sa, dstp_sa, zf)
    if isinstance(p1, (tuple, list)):
        p1 = p1[0]

    def l1_body(p_ref, c_ref, xa1_ref, wl, wr, bl_ref, wo_ref, bo_ref, o_ref):
        mean = p_ref[...] / jnp.maximum(c_ref[...], 1.0)
        h = jnp.maximum(
            _mmt(mean, wl[...]) + bl_ref[...] + _mmt(xa1_ref[...], wr[...]),
            0.0)
        o_ref[...] = (jnp.sum(h * wo_ref[...], axis=1, keepdims=True)
                      + bo_ref[0, 0])

    pred = pl.pallas_call(
        l1_body,
        grid=(nb,),
        in_specs=[
            rowspec, rowspec, rowspec, wspec, wspec, bspec,
            pl.BlockSpec((1, d), lambda i: (0, 0)),
            pl.BlockSpec((1, 1), lambda i: (0, 0)),
        ],
        out_specs=pl.BlockSpec((BM, 1), lambda i: (i, 0)),
        out_shape=jax.ShapeDtypeStruct((n, 1), jnp.float32),
    )(p1, cnt_sa, xa1, Wl_sa_1, Wr_sa_1, bl_sa_1.reshape(1, d),
      Wo, bo.reshape(1, 1))

    return pred[:, 0]


# speedup vs baseline: 1.5343x; 1.2718x over previous
import functools

import jax
import jax.numpy as jnp
from jax import lax
from jax.experimental import pallas as pl
from jax.experimental.pallas import tpu as pltpu
from jax.experimental.pallas import tpu_sc as plsc

NS = 16
NL = 16
K = 128
BM = 1000
NH = 3456
NP = 3


def _mmt(a, w):
    return lax.dot_general(a, w, (((1,), (1,)), ((), ())),
                           preferred_element_type=jnp.float32)


def _cnt_pad(n_nodes):
    gran = NS * NL
    return -(-(n_nodes + 1) // gran) * gran


def _make_segsum(n_nodes, d, nch, with_counts):
    stripe = NH // NS
    npad = _cnt_pad(n_nodes)
    cpt = npad // NS

    if with_counts:
        outs = [jax.ShapeDtypeStruct((NP * NH, d), jnp.float32),
                jax.ShapeDtypeStruct((NP * NH, d), jnp.float32),
                jax.ShapeDtypeStruct((2 * NS * npad,), jnp.float32)]
    else:
        outs = [jax.ShapeDtypeStruct((NP * NH, d), jnp.float32)]
    scratch = [
        pltpu.VMEM((nch, K), jnp.int32),
        pltpu.VMEM((nch, K), jnp.int32),
        pltpu.VMEM((K, d), jnp.float32),
        pltpu.VMEM((K, d), jnp.float32),
        pltpu.VMEM((K, d), jnp.float32),
        pltpu.VMEM_SHARED((NH + 8, d), jnp.float32),
    ]
    if with_counts:
        scratch += [
            pltpu.VMEM((npad,), jnp.float32),
        ]
    scratch += [pltpu.SemaphoreType.DMA] * 6

    mesh = plsc.VectorSubcoreMesh(core_axis_name="c", subcore_axis_name="s",
                                  num_cores=1, num_subcores=NS)

    @functools.partial(
        pl.kernel, out_type=tuple(outs), mesh=mesh, scratch_types=scratch,
        compiler_params=pltpu.CompilerParams(needs_layout_passes=False))
    def seg(*refs):
        if with_counts:
            (x0, src0, dstp0, dstg0, x1, src1, dstp1, dstg1, zf, zc,
             pout0, pout1, craw, srcv, dstv, r0, r1, r2, accum,
             cntv, *sems) = refs
            types = ((x0, src0, dstp0, dstg0, pout0),
                     (x1, src1, dstp1, dstg1, pout1))
        else:
            (x0, src0, dstp0, zf, pout0,
             srcv, dstv, r0, r1, r2, accum, *sems) = refs
            types = ((x0, src0, dstp0, None, pout0),)
        s = lax.axis_index("s")
        rows = (r0, r1, r2)
        gsem = sems[0:3]
        ssem = sems[3:6]

        def wait_g(sl):
            pltpu.make_async_copy(x0.at[srcv.at[0]], rows[sl],
                                  gsem[sl]).wait()

        def wait_s(sl):
            pltpu.make_async_copy(rows[sl], accum.at[dstv.at[0]],
                                  ssem[sl]).wait()

        for t, (x, src, dstp, dstg, pout) in enumerate(types):
            pltpu.sync_copy(src.at[s], srcv)
            for p in range(NP):
                pltpu.sync_copy(dstp.at[p, s], dstv)
                pltpu.sync_copy(zf, accum.at[pl.ds(s * stripe, stripe)])

                @pl.when(s == 0)
                def _():
                    pltpu.sync_copy(zf.at[pl.ds(0, 8)],
                                    accum.at[pl.ds(NH, 8)])

                plsc.subcore_barrier()

                pltpu.async_copy(x.at[srcv.at[0]], rows[0], gsem[0])
                pltpu.async_copy(x.at[srcv.at[1]], rows[1], gsem[1])

                @pl.loop(0, nch - 2, step=2)
                def _(g):
                    wait_g(0)
                    pltpu.async_copy(rows[0], accum.at[dstv.at[g]], ssem[0],
                                     add=True)
                    wait_g(1)
                    wait_s(0)
                    pltpu.async_copy(x.at[srcv.at[g + 2]], rows[0], gsem[0])
                    pltpu.async_copy(rows[1], accum.at[dstv.at[g + 1]],
                                     ssem[1], add=True)
                    wait_s(1)
                    pltpu.async_copy(x.at[srcv.at[g + 3]], rows[1], gsem[1])

                wait_g(0)
                pltpu.async_copy(rows[0], accum.at[dstv.at[nch - 2]],
                                 ssem[0], add=True)
                wait_g(1)
                wait_s(0)
                pltpu.async_copy(rows[1], accum.at[dstv.at[nch - 1]],
                                 ssem[1], add=True)
                wait_s(1)
                plsc.subcore_barrier()
                pltpu.sync_copy(accum.at[pl.ds(s * stripe, stripe)],
                                pout.at[pl.ds(p * NH + s * stripe, stripe)])

            if with_counts:
                pltpu.sync_copy(zc, cntv)
                pltpu.sync_copy(dstg.at[s], dstv)
                ones = jnp.full((NL,), 1.0, jnp.float32)

                @pl.loop(0, nch)
                def _(ci):
                    for i in range(K // NL):
                        idx16 = dstv[ci, pl.ds(i * NL, NL)]
                        plsc.addupdate_scatter(cntv, [idx16], ones)

                pltpu.sync_copy(
                    cntv, craw.at[pl.ds((t * NS + s) * npad, npad)])

    return seg


def _make_cnt_reduce(n_nodes):
    npad = _cnt_pad(n_nodes)
    cpt = npad // NS

    mesh = plsc.VectorSubcoreMesh(core_axis_name="c", subcore_axis_name="s",
                                  num_cores=1, num_subcores=NS)

    @functools.partial(
        pl.kernel,
        out_type=jax.ShapeDtypeStruct((2 * npad,), jnp.float32),
        mesh=mesh,
        scratch_types=[
            pltpu.VMEM((NS, cpt), jnp.float32),
            pltpu.VMEM((cpt,), jnp.float32),
            pltpu.SemaphoreType.DMA,
        ],
        compiler_params=pltpu.CompilerParams(needs_layout_passes=False))
    def red(craw, cout, cstripe, cred, sem):
        s = lax.axis_index("s")
        for t in range(2):
            descs = [
                pltpu.async_copy(
                    craw.at[pl.ds((t * NS + r) * npad + s * cpt, cpt)],
                    cstripe.at[r], sem)
                for r in range(NS)
            ]
            for de in descs:
                de.wait()
            for j in range(cpt // NL):
                acc = cstripe[0, pl.ds(j * NL, NL)]
                for r in range(1, NS):
                    acc = acc + cstripe[r, pl.ds(j * NL, NL)]
                cred[pl.ds(j * NL, NL)] = acc
            pltpu.sync_copy(cred, cout.at[pl.ds(t * npad + s * cpt, cpt)])

    return red


def _prep_edges(src, dst, nch, n_nodes):
    tot = NS * nch * K
    padn = tot - src.shape[0]
    if padn:
        src = jnp.concatenate([src, jnp.zeros((padn,), jnp.int32)])
        dst = jnp.concatenate([dst, jnp.full((padn,), n_nodes, jnp.int32)])
    shp = (NS, nch, K)
    dummy = NH + (jnp.arange(tot, dtype=jnp.int32) & 7)
    dstp = jnp.stack([
        jnp.where((dst >= p * NH) & (dst < min((p + 1) * NH, n_nodes)),
                  dst - p * NH, dummy).reshape(shp)
        for p in range(NP)
    ])
    return src.reshape(shp), dstp, dst.reshape(shp)


def kernel(x_students, x_assignments, ei_sa, ei_as, Wp_s, bp_s, Wp_a, bp_a,
           Wl_sa_0, bl_sa_0, Wr_sa_0, Wl_as_0, bl_as_0, Wr_as_0,
           Wl_sa_1, bl_sa_1, Wr_sa_1, Wl_as_1, bl_as_1, Wr_as_1,
           Wo, bo):
    n, d = x_students.shape
    e = ei_sa.shape[1]
    nb = n // BM
    npad = _cnt_pad(n)

    def proj_body(xs_ref, xa_ref, ws_ref, wa_ref, bs_ref, ba_ref,
                  os_ref, oa_ref):
        os_ref[...] = jnp.maximum(_mmt(xs_ref[...], ws_ref[...]) + bs_ref[...],
                                  0.0)
        oa_ref[...] = jnp.maximum(_mmt(xa_ref[...], wa_ref[...]) + ba_ref[...],
                                  0.0)

    wspec = pl.BlockSpec((d, d), lambda i: (0, 0))
    bspec = pl.BlockSpec((1, d), lambda i: (0, 0))
    rowspec = pl.BlockSpec((BM, d), lambda i: (i, 0))
    fspec = jax.ShapeDtypeStruct((n, d), jnp.float32)
    xs, xa = pl.pallas_call(
        proj_body,
        grid=(nb,),
        in_specs=[rowspec, rowspec, wspec, wspec, bspec, bspec],
        out_specs=[rowspec, rowspec],
        out_shape=[fspec, fspec],
    )(x_students, x_assignments, Wp_s, Wp_a,
      bp_s.reshape(1, d), bp_a.reshape(1, d))

    zf = jnp.zeros((NH // NS, d), jnp.float32)
    zc = jnp.zeros((npad,), jnp.float32)

    nch = -(-e // (NS * K))
    nch += nch % 2
    src_sa, dstp_sa, dstg_sa = _prep_edges(ei_sa[0], ei_sa[1], nch, n)
    src_as, dstp_as, dstg_as = _prep_edges(ei_as[0], ei_as[1], nch, n)
    seg_c = _make_segsum(n, d, nch, with_counts=True)
    p_sa, p_as, craw = seg_c(
        xs, src_sa, dstp_sa, dstg_sa, xa, src_as, dstp_as, dstg_as, zf, zc)
    c_all = _make_cnt_reduce(n)(craw)
    cnt_sa = jnp.broadcast_to(c_all[:n, None], (n, d))
    cnt_as = jnp.broadcast_to(c_all[npad:npad + n, None], (n, d))

    def l0_body(psa_ref, csa_ref, pas_ref, cas_ref, xs_ref, xa_ref,
                wl_sa, wr_sa, wl_as, wr_as, bsa_ref, bas_ref,
                xa1_ref, xs1_ref):
        mean_sa = psa_ref[...] / jnp.maximum(csa_ref[...], 1.0)
        xa1_ref[...] = jnp.maximum(
            _mmt(mean_sa, wl_sa[...]) + bsa_ref[...]
            + _mmt(xa_ref[...], wr_sa[...]), 0.0)
        mean_as = pas_ref[...] / jnp.maximum(cas_ref[...], 1.0)
        xs1_ref[...] = jnp.maximum(
            _mmt(mean_as, wl_as[...]) + bas_ref[...]
            + _mmt(xs_ref[...], wr_as[...]), 0.0)

    xa1, xs1 = pl.pallas_call(
        l0_body,
        grid=(nb,),
        in_specs=[rowspec, rowspec, rowspec, rowspec, rowspec, rowspec,
                  wspec, wspec, wspec, wspec, bspec, bspec],
        out_specs=[rowspec, rowspec],
        out_shape=[fspec, fspec],
    )(p_sa, cnt_sa, p_as, cnt_as, xs, xa, Wl_sa_0, Wr_sa_0, Wl_as_0, Wr_as_0,
      bl_sa_0.reshape(1, d), bl_as_0.reshape(1, d))

    seg_n = _make_segsum(n, d, nch, with_counts=False)
    p1 = seg_n(xs1, src_sa, dstp_sa, zf)
    if isinstance(p1, (tuple, list)):
        p1 = p1[0]

    def l1_body(p_ref, c_ref, xa1_ref, wl, wr, bl_ref, wo_ref, bo_ref, o_ref):
        mean = p_ref[...] / jnp.maximum(c_ref[...], 1.0)
        h = jnp.maximum(
            _mmt(mean, wl[...]) + bl_ref[...] + _mmt(xa1_ref[...], wr[...]),
            0.0)
        o_ref[...] = (jnp.sum(h * wo_ref[...], axis=1, keepdims=True)
                      + bo_ref[0, 0])

    pred = pl.pallas_call(
        l1_body,
        grid=(nb,),
        in_specs=[
            rowspec, rowspec, rowspec, wspec, wspec, bspec,
            pl.BlockSpec((1, d), lambda i: (0, 0)),
            pl.BlockSpec((1, 1), lambda i: (0, 0)),
        ],
        out_specs=pl.BlockSpec((BM, 1), lambda i: (i, 0)),
        out_shape=jax.ShapeDtypeStruct((n, 1), jnp.float32),
    )(p1, cnt_sa, xa1, Wl_sa_1, Wr_sa_1, bl_sa_1.reshape(1, d),
      Wo, bo.reshape(1, 1))

    return pred[:, 0]


# speedup vs baseline: 1.5428x; 1.0056x over previous
import functools

import jax
import jax.numpy as jnp
from jax import lax
from jax.experimental import pallas as pl
from jax.experimental.pallas import tpu as pltpu
from jax.experimental.pallas import tpu_sc as plsc

NS = 16
NL = 16
K = 128
BM = 1000
NH = 3456
NP = 3


def _mmt(a, w):
    return lax.dot_general(a, w, (((1,), (1,)), ((), ())),
                           preferred_element_type=jnp.float32)


def _cnt_pad(n_nodes):
    gran = NS * NL
    return -(-(n_nodes + 1) // gran) * gran


def _make_segsum(n_nodes, d, nch, with_counts):
    stripe = NH // NS
    npad = _cnt_pad(n_nodes)
    cpt = npad // NS

    if with_counts:
        outs = [jax.ShapeDtypeStruct((NP * NH, d), jnp.float32),
                jax.ShapeDtypeStruct((NP * NH, d), jnp.float32),
                jax.ShapeDtypeStruct((2 * NS * npad,), jnp.float32)]
    else:
        outs = [jax.ShapeDtypeStruct((NP * NH, d), jnp.float32)]
    scratch = [
        pltpu.VMEM((nch, K), jnp.int32),
        pltpu.VMEM((nch, K), jnp.int32),
        pltpu.VMEM((K, d), jnp.float32),
        pltpu.VMEM((K, d), jnp.float32),
        pltpu.VMEM((K, d), jnp.float32),
        pltpu.VMEM_SHARED((NH + 16, d), jnp.float32),
    ]
    if with_counts:
        scratch += [
            pltpu.VMEM((npad,), jnp.float32),
        ]
    scratch += [pltpu.SemaphoreType.DMA] * 6

    mesh = plsc.VectorSubcoreMesh(core_axis_name="c", subcore_axis_name="s",
                                  num_cores=1, num_subcores=NS)

    @functools.partial(
        pl.kernel, out_type=tuple(outs), mesh=mesh, scratch_types=scratch,
        compiler_params=pltpu.CompilerParams(needs_layout_passes=False))
    def seg(*refs):
        if with_counts:
            (x0, src0, dstp0, dstg0, x1, src1, dstp1, dstg1, zf, zc,
             pout0, pout1, craw, srcv, dstv, r0, r1, r2, accum,
             cntv, *sems) = refs
            types = ((x0, src0, dstp0, dstg0, pout0),
                     (x1, src1, dstp1, dstg1, pout1))
        else:
            (x0, src0, dstp0, zf, pout0,
             srcv, dstv, r0, r1, r2, accum, *sems) = refs
            types = ((x0, src0, dstp0, None, pout0),)
        s = lax.axis_index("s")
        rows = (r0, r1, r2)
        gsem = sems[0:3]
        ssem = sems[3:6]

        def wait_g(sl):
            pltpu.make_async_copy(x0.at[srcv.at[0]], rows[sl],
                                  gsem[sl]).wait()

        def wait_s(sl):
            pltpu.make_async_copy(rows[sl], accum.at[dstv.at[0]],
                                  ssem[sl]).wait()

        for t, (x, src, dstp, dstg, pout) in enumerate(types):
            pltpu.sync_copy(src.at[s], srcv)
            for p in range(NP):
                pltpu.sync_copy(dstp.at[p, s], dstv)
                pltpu.sync_copy(zf, accum.at[pl.ds(s * stripe, stripe)])

                @pl.when(s == 0)
                def _():
                    pltpu.sync_copy(zf.at[pl.ds(0, 16)],
                                    accum.at[pl.ds(NH, 16)])

                plsc.subcore_barrier()

                pltpu.async_copy(x.at[srcv.at[0]], rows[0], gsem[0])
                pltpu.async_copy(x.at[srcv.at[1]], rows[1], gsem[1])

                @pl.loop(0, nch - 2, step=2)
                def _(g):
                    wait_g(0)
                    pltpu.async_copy(rows[0], accum.at[dstv.at[g]], ssem[0],
                                     add=True)
                    wait_g(1)
                    wait_s(0)
                    pltpu.async_copy(x.at[srcv.at[g + 2]], rows[0], gsem[0])
                    pltpu.async_copy(rows[1], accum.at[dstv.at[g + 1]],
                                     ssem[1], add=True)
                    wait_s(1)
                    pltpu.async_copy(x.at[srcv.at[g + 3]], rows[1], gsem[1])

                wait_g(0)
                pltpu.async_copy(rows[0], accum.at[dstv.at[nch - 2]],
                                 ssem[0], add=True)
                wait_g(1)
                wait_s(0)
                pltpu.async_copy(rows[1], accum.at[dstv.at[nch - 1]],
                                 ssem[1], add=True)
                wait_s(1)
                plsc.subcore_barrier()
                pltpu.sync_copy(accum.at[pl.ds(s * stripe, stripe)],
                                pout.at[pl.ds(p * NH + s * stripe, stripe)])

            if with_counts:
                pltpu.sync_copy(zc, cntv)
                pltpu.sync_copy(dstg.at[s], dstv)
                ones = jnp.full((NL,), 1.0, jnp.float32)

                @pl.loop(0, nch)
                def _(ci):
                    for i in range(K // NL):
                        idx16 = dstv[ci, pl.ds(i * NL, NL)]
                        plsc.addupdate_scatter(cntv, [idx16], ones)

                pltpu.sync_copy(
                    cntv, craw.at[pl.ds((t * NS + s) * npad, npad)])

    return seg


def _make_cnt_reduce(n_nodes):
    npad = _cnt_pad(n_nodes)
    cpt = npad // NS

    mesh = plsc.VectorSubcoreMesh(core_axis_name="c", subcore_axis_name="s",
                                  num_cores=1, num_subcores=NS)

    @functools.partial(
        pl.kernel,
        out_type=jax.ShapeDtypeStruct((2 * npad,), jnp.float32),
        mesh=mesh,
        scratch_types=[
            pltpu.VMEM((NS, cpt), jnp.float32),
            pltpu.VMEM((cpt,), jnp.float32),
            pltpu.SemaphoreType.DMA,
        ],
        compiler_params=pltpu.CompilerParams(needs_layout_passes=False))
    def red(craw, cout, cstripe, cred, sem):
        s = lax.axis_index("s")
        for t in range(2):
            descs = [
                pltpu.async_copy(
                    craw.at[pl.ds((t * NS + r) * npad + s * cpt, cpt)],
                    cstripe.at[r], sem)
                for r in range(NS)
            ]
            for de in descs:
                de.wait()
            for j in range(cpt // NL):
                acc = cstripe[0, pl.ds(j * NL, NL)]
                for r in range(1, NS):
                    acc = acc + cstripe[r, pl.ds(j * NL, NL)]
                cred[pl.ds(j * NL, NL)] = acc
            pltpu.sync_copy(cred, cout.at[pl.ds(t * npad + s * cpt, cpt)])

    return red


def _prep_edges(src, dst, nch, n_nodes):
    tot = NS * nch * K
    padn = tot - src.shape[0]
    if padn:
        src = jnp.concatenate([src, jnp.zeros((padn,), jnp.int32)])
        dst = jnp.concatenate([dst, jnp.full((padn,), n_nodes, jnp.int32)])
    shp = (NS, nch, K)
    dummy = NH + (jnp.arange(tot, dtype=jnp.int32) & 15)
    dstp = jnp.stack([
        jnp.where((dst >= p * NH) & (dst < min((p + 1) * NH, n_nodes)),
                  dst - p * NH, dummy).reshape(shp)
        for p in range(NP)
    ])
    return src.reshape(shp), dstp, dst.reshape(shp)


def kernel(x_students, x_assignments, ei_sa, ei_as, Wp_s, bp_s, Wp_a, bp_a,
           Wl_sa_0, bl_sa_0, Wr_sa_0, Wl_as_0, bl_as_0, Wr_as_0,
           Wl_sa_1, bl_sa_1, Wr_sa_1, Wl_as_1, bl_as_1, Wr_as_1,
           Wo, bo):
    n, d = x_students.shape
    e = ei_sa.shape[1]
    nb = n // BM
    npad = _cnt_pad(n)

    def proj_body(xs_ref, xa_ref, ws_ref, wa_ref, bs_ref, ba_ref,
                  os_ref, oa_ref):
        os_ref[...] = jnp.maximum(_mmt(xs_ref[...], ws_ref[...]) + bs_ref[...],
                                  0.0)
        oa_ref[...] = jnp.maximum(_mmt(xa_ref[...], wa_ref[...]) + ba_ref[...],
                                  0.0)

    wspec = pl.BlockSpec((d, d), lambda i: (0, 0))
    bspec = pl.BlockSpec((1, d), lambda i: (0, 0))
    rowspec = pl.BlockSpec((BM, d), lambda i: (i, 0))
    fspec = jax.ShapeDtypeStruct((n, d), jnp.float32)
    xs, xa = pl.pallas_call(
        proj_body,
        grid=(nb,),
        in_specs=[rowspec, rowspec, wspec, wspec, bspec, bspec],
        out_specs=[rowspec, rowspec],
        out_shape=[fspec, fspec],
    )(x_students, x_assignments, Wp_s, Wp_a,
      bp_s.reshape(1, d), bp_a.reshape(1, d))

    zf = jnp.zeros((NH // NS, d), jnp.float32)
    zc = jnp.zeros((npad,), jnp.float32)

    nch = -(-e // (NS * K))
    nch += nch % 2
    src_sa, dstp_sa, dstg_sa = _prep_edges(ei_sa[0], ei_sa[1], nch, n)
    src_as, dstp_as, dstg_as = _prep_edges(ei_as[0], ei_as[1], nch, n)
    seg_c = _make_segsum(n, d, nch, with_counts=True)
    p_sa, p_as, craw = seg_c(
        xs, src_sa, dstp_sa, dstg_sa, xa, src_as, dstp_as, dstg_as, zf, zc)
    c_all = _make_cnt_reduce(n)(craw)
    cnt_sa = jnp.broadcast_to(c_all[:n, None], (n, d))
    cnt_as = jnp.broadcast_to(c_all[npad:npad + n, None], (n, d))

    def l0_body(psa_ref, csa_ref, pas_ref, cas_ref, xs_ref, xa_ref,
                wl_sa, wr_sa, wl_as, wr_as, bsa_ref, bas_ref,
                xa1_ref, xs1_ref):
        mean_sa = psa_ref[...] / jnp.maximum(csa_ref[...], 1.0)
        xa1_ref[...] = jnp.maximum(
            _mmt(mean_sa, wl_sa[...]) + bsa_ref[...]
            + _mmt(xa_ref[...], wr_sa[...]), 0.0)
        mean_as = pas_ref[...] / jnp.maximum(cas_ref[...], 1.0)
        xs1_ref[...] = jnp.maximum(
            _mmt(mean_as, wl_as[...]) + bas_ref[...]
            + _mmt(xs_ref[...], wr_as[...]), 0.0)

    xa1, xs1 = pl.pallas_call(
        l0_body,
        grid=(nb,),
        in_specs=[rowspec, rowspec, rowspec, rowspec, rowspec, rowspec,
                  wspec, wspec, wspec, wspec, bspec, bspec],
        out_specs=[rowspec, rowspec],
        out_shape=[fspec, fspec],
    )(p_sa, cnt_sa, p_as, cnt_as, xs, xa, Wl_sa_0, Wr_sa_0, Wl_as_0, Wr_as_0,
      bl_sa_0.reshape(1, d), bl_as_0.reshape(1, d))

    seg_n = _make_segsum(n, d, nch, with_counts=False)
    p1 = seg_n(xs1, src_sa, dstp_sa, zf)
    if isinstance(p1, (tuple, list)):
        p1 = p1[0]

    def l1_body(p_ref, c_ref, xa1_ref, wl, wr, bl_ref, wo_ref, bo_ref, o_ref):
        mean = p_ref[...] / jnp.maximum(c_ref[...], 1.0)
        h = jnp.maximum(
            _mmt(mean, wl[...]) + bl_ref[...] + _mmt(xa1_ref[...], wr[...]),
            0.0)
        o_ref[...] = (jnp.sum(h * wo_ref[...], axis=1, keepdims=True)
                      + bo_ref[0, 0])

    pred = pl.pallas_call(
        l1_body,
        grid=(nb,),
        in_specs=[
            rowspec, rowspec, rowspec, wspec, wspec, bspec,
            pl.BlockSpec((1, d), lambda i: (0, 0)),
            pl.BlockSpec((1, 1), lambda i: (0, 0)),
        ],
        out_specs=pl.BlockSpec((BM, 1), lambda i: (i, 0)),
        out_shape=jax.ShapeDtypeStruct((n, 1), jnp.float32),
    )(p1, cnt_sa, xa1, Wl_sa_1, Wr_sa_1, bl_sa_1.reshape(1, d),
      Wo, bo.reshape(1, 1))

    return pred[:, 0]
